# baseline (device time: 653178 ns/iter reference)
import jax
import jax.numpy as jnp
from jax import lax
from jax.experimental import pallas as pl
from jax.experimental.pallas import tpu as pltpu

B, S, H_LOC, D = 4, 1024, 16, 128
K = H_LOC * D
N = 4096
S_HALF = S // 2
N_CHUNK = 1024


def kernel(O, Wo):
    x = O.reshape(B * S, K)
    w = Wo.astype(jnp.bfloat16)

    def body(x_hbm, w_ref, out_hbm, x_tiles, send_buf, recv_buf, out_stage,
             x_sems, send_sems, recv_sems, out_sems, credit_sem):
        my_x = lax.axis_index("x")
        my_y = lax.axis_index("y")
        my_z = lax.axis_index("z")
        peer = (1 - my_x, my_y, my_z)

        def load_half(b, which, slot):
            return pltpu.make_async_copy(
                x_hbm.at[pl.ds(b * S + which * S_HALF, S_HALF), :],
                x_tiles.at[slot],
                x_sems.at[slot],
            )

        def load_theirs(b):
            return load_half(b, 1 - my_x, 0)

        def load_mine(b):
            return load_half(b, my_x, 1)

        load_theirs(0).start()
        load_mine(0).start()

        barrier_sem = pltpu.get_barrier_semaphore()
        pl.semaphore_signal(
            barrier_sem, inc=1, device_id=peer,
            device_id_type=pl.DeviceIdType.MESH,
        )
        pl.semaphore_wait(barrier_sem, 1)

        rdmas = [None] * B
        out_dmas = [None] * B
        for b in range(B):
            slot = b % 2

            load_theirs(b).wait()
            if b >= 2:
                rdmas[b - 2].wait_send()
            xt = x_tiles[0].astype(jnp.bfloat16)
            for nj in range(N // N_CHUNK):
                cols = slice(nj * N_CHUNK, (nj + 1) * N_CHUNK)
                send_buf[slot, :, cols] = jnp.dot(
                    xt, w_ref[:, cols], preferred_element_type=jnp.float32
                ).astype(jnp.bfloat16)
            if b + 1 < B:
                load_theirs(b + 1).start()

            if b >= 2:
                pl.semaphore_wait(credit_sem, 1)
            rdma = pltpu.make_async_remote_copy(
                src_ref=send_buf.at[slot],
                dst_ref=recv_buf.at[slot],
                send_sem=send_sems.at[slot],
                recv_sem=recv_sems.at[slot],
                device_id=peer,
                device_id_type=pl.DeviceIdType.MESH,
            )
            rdma.start()
            rdmas[b] = rdma

            load_mine(b).wait()
            if b >= 2:
                out_dmas[b - 2].wait()
            xm = x_tiles[1].astype(jnp.bfloat16)
            for nj in range(N // N_CHUNK):
                cols = slice(nj * N_CHUNK, (nj + 1) * N_CHUNK)
                out_stage[slot, :, cols] = jnp.dot(
                    xm, w_ref[:, cols], preferred_element_type=jnp.float32
                )
            if b + 1 < B:
                load_mine(b + 1).start()

            if b > 0:
                pslot = (b - 1) % 2
                rdmas[b - 1].wait_recv()
                out_stage[pslot] = out_stage[pslot] + recv_buf[pslot].astype(
                    jnp.float32
                )
                if b - 1 < B - 2:
                    pl.semaphore_signal(
                        credit_sem, inc=1, device_id=peer,
                        device_id_type=pl.DeviceIdType.MESH,
                    )
                dma = pltpu.make_async_copy(
                    out_stage.at[pslot], out_hbm.at[b - 1], out_sems.at[pslot]
                )
                dma.start()
                out_dmas[b - 1] = dma

        lslot = (B - 1) % 2
        rdmas[B - 1].wait_recv()
        out_stage[lslot] = out_stage[lslot] + recv_buf[lslot].astype(
            jnp.float32
        )
        dma = pltpu.make_async_copy(
            out_stage.at[lslot], out_hbm.at[B - 1], out_sems.at[lslot]
        )
        dma.start()
        out_dmas[B - 2].wait()
        dma.wait()
        rdmas[B - 2].wait_send()
        rdmas[B - 1].wait_send()

    return pl.pallas_call(
        body,
        out_shape=jax.ShapeDtypeStruct((B, S_HALF, N), jnp.float32),
        in_specs=[
            pl.BlockSpec(memory_space=pltpu.MemorySpace.HBM),
            pl.BlockSpec(memory_space=pltpu.VMEM),
        ],
        out_specs=pl.BlockSpec(memory_space=pltpu.MemorySpace.HBM),
        scratch_shapes=[
            pltpu.VMEM((2, S_HALF, K), jnp.float32),
            pltpu.VMEM((2, S_HALF, N), jnp.bfloat16),
            pltpu.VMEM((2, S_HALF, N), jnp.bfloat16),
            pltpu.VMEM((2, S_HALF, N), jnp.float32),
            pltpu.SemaphoreType.DMA((2,)),
            pltpu.SemaphoreType.DMA((2,)),
            pltpu.SemaphoreType.DMA((2,)),
            pltpu.SemaphoreType.DMA((2,)),
            pltpu.SemaphoreType.REGULAR,
        ],
        compiler_params=pltpu.CompilerParams(
            collective_id=0,
            vmem_limit_bytes=96 * 1024 * 1024,
        ),
    )(x, w)


# device time: 246183 ns/iter; 2.6532x vs baseline; 2.6532x over previous
import jax
import jax.numpy as jnp
from jax import lax
from jax.experimental import pallas as pl
from jax.experimental.pallas import tpu as pltpu

B, S, H_LOC, D = 4, 1024, 16, 128
K = H_LOC * D
N = 4096
S_HALF = S // 2
N_CHUNK = 1024


def kernel(O, Wo):
    w = Wo.astype(jnp.bfloat16)

    def body(x_hbm, w_ref, out_hbm, x_tiles, send_buf, recv_buf, out_stage,
             x_sems, send_sems, recv_sems, out_sems, credit_sem):
        my_x = lax.axis_index("x")
        my_y = lax.axis_index("y")
        my_z = lax.axis_index("z")
        peer = (1 - my_x, my_y, my_z)

        def load_half(b, which, slot):
            s0 = which * S_HALF
            return [
                pltpu.make_async_copy(
                    x_hbm.at[b, pl.ds(s0, S_HALF), h, :],
                    x_tiles.at[slot, :, pl.ds(h * D, D)],
                    x_sems.at[slot],
                )
                for h in range(H_LOC)
            ]

        def load_theirs(b):
            return load_half(b, 1 - my_x, 0)

        def load_mine(b):
            return load_half(b, my_x, 1)

        def start_all(copies):
            for c in copies:
                c.start()

        def wait_all(copies):
            for c in copies:
                c.wait()

        start_all(load_theirs(0))
        start_all(load_mine(0))

        barrier_sem = pltpu.get_barrier_semaphore()
        pl.semaphore_signal(
            barrier_sem, inc=1, device_id=peer,
            device_id_type=pl.DeviceIdType.MESH,
        )
        pl.semaphore_wait(barrier_sem, 1)

        rdmas = [None] * B
        out_dmas = [None] * B
        for b in range(B):
            slot = b % 2

            wait_all(load_theirs(b))
            if b >= 2:
                rdmas[b - 2].wait_send()
            xt = x_tiles[0].astype(jnp.bfloat16)
            for nj in range(N // N_CHUNK):
                cols = slice(nj * N_CHUNK, (nj + 1) * N_CHUNK)
                send_buf[slot, :, cols] = jnp.dot(
                    xt, w_ref[:, cols], preferred_element_type=jnp.float32
                ).astype(jnp.bfloat16)
            if b + 1 < B:
                start_all(load_theirs(b + 1))

            if b >= 2:
                pl.semaphore_wait(credit_sem, 1)
            rdma = pltpu.make_async_remote_copy(
                src_ref=send_buf.at[slot],
                dst_ref=recv_buf.at[slot],
                send_sem=send_sems.at[slot],
                recv_sem=recv_sems.at[slot],
                device_id=peer,
                device_id_type=pl.DeviceIdType.MESH,
            )
            rdma.start()
            rdmas[b] = rdma

            wait_all(load_mine(b))
            if b >= 2:
                out_dmas[b - 2].wait()
            xm = x_tiles[1].astype(jnp.bfloat16)
            for nj in range(N // N_CHUNK):
                cols = slice(nj * N_CHUNK, (nj + 1) * N_CHUNK)
                out_stage[slot, :, cols] = jnp.dot(
                    xm, w_ref[:, cols], preferred_element_type=jnp.float32
                )
            if b + 1 < B:
                start_all(load_mine(b + 1))

            if b > 0:
                pslot = (b - 1) % 2
                rdmas[b - 1].wait_recv()
                out_stage[pslot] = out_stage[pslot] + recv_buf[pslot].astype(
                    jnp.float32
                )
                if b - 1 < B - 2:
                    pl.semaphore_signal(
                        credit_sem, inc=1, device_id=peer,
                        device_id_type=pl.DeviceIdType.MESH,
                    )
                dma = pltpu.make_async_copy(
                    out_stage.at[pslot], out_hbm.at[b - 1], out_sems.at[pslot]
                )
                dma.start()
                out_dmas[b - 1] = dma

        lslot = (B - 1) % 2
        rdmas[B - 1].wait_recv()
        out_stage[lslot] = out_stage[lslot] + recv_buf[lslot].astype(
            jnp.float32
        )
        dma = pltpu.make_async_copy(
            out_stage.at[lslot], out_hbm.at[B - 1], out_sems.at[lslot]
        )
        dma.start()
        out_dmas[B - 2].wait()
        dma.wait()
        rdmas[B - 2].wait_send()
        rdmas[B - 1].wait_send()

    return pl.pallas_call(
        body,
        out_shape=jax.ShapeDtypeStruct((B, S_HALF, N), jnp.float32),
        in_specs=[
            pl.BlockSpec(memory_space=pltpu.MemorySpace.HBM),
            pl.BlockSpec(memory_space=pltpu.VMEM),
        ],
        out_specs=pl.BlockSpec(memory_space=pltpu.MemorySpace.HBM),
        scratch_shapes=[
            pltpu.VMEM((2, S_HALF, K), jnp.float32),
            pltpu.VMEM((2, S_HALF, N), jnp.bfloat16),
            pltpu.VMEM((2, S_HALF, N), jnp.bfloat16),
            pltpu.VMEM((2, S_HALF, N), jnp.float32),
            pltpu.SemaphoreType.DMA((2,)),
            pltpu.SemaphoreType.DMA((2,)),
            pltpu.SemaphoreType.DMA((2,)),
            pltpu.SemaphoreType.DMA((2,)),
            pltpu.SemaphoreType.REGULAR,
        ],
        compiler_params=pltpu.CompilerParams(
            collective_id=0,
            vmem_limit_bytes=96 * 1024 * 1024,
        ),
    )(O, w)


# device time: 238706 ns/iter; 2.7363x vs baseline; 1.0313x over previous
import jax
import jax.numpy as jnp
from jax import lax
from jax.experimental import pallas as pl
from jax.experimental.pallas import tpu as pltpu

B, S, H_LOC, D = 4, 1024, 16, 128
K = H_LOC * D
N = 4096
S_HALF = S // 2
N_CHUNK = 1024


def kernel(O, Wo):
    w = Wo.astype(jnp.bfloat16)

    def body(x_hbm, w_ref, out_hbm, x_tiles, send_buf, recv_buf, out_stage,
             x_sems, send_sems, recv_sems, out_sems, credit_sem):
        my_x = lax.axis_index("x")
        my_y = lax.axis_index("y")
        my_z = lax.axis_index("z")
        peer = (1 - my_x, my_y, my_z)

        def load_half(b, which, slot):
            s0 = which * S_HALF
            return [
                pltpu.make_async_copy(
                    x_hbm.at[b, pl.ds(s0, S_HALF), h, :],
                    x_tiles.at[slot, :, pl.ds(h * D, D)],
                    x_sems.at[slot],
                )
                for h in range(H_LOC)
            ]

        def load_theirs(b):
            return load_half(b, 1 - my_x, 0)

        def load_mine(b):
            return load_half(b, my_x, 1)

        def start_all(copies):
            for c in copies:
                c.start()

        def wait_all(copies):
            for c in copies:
                c.wait()

        start_all(load_theirs(0))
        start_all(load_mine(0))

        barrier_sem = pltpu.get_barrier_semaphore()
        pl.semaphore_signal(
            barrier_sem, inc=1, device_id=peer,
            device_id_type=pl.DeviceIdType.MESH,
        )
        pl.semaphore_wait(barrier_sem, 1)

        NCH = N // N_CHUNK

        def chunk_rdma(slot, nj):
            cols = pl.ds(nj * N_CHUNK, N_CHUNK)
            return pltpu.make_async_remote_copy(
                src_ref=send_buf.at[slot, :, cols],
                dst_ref=recv_buf.at[slot, :, cols],
                send_sem=send_sems.at[slot, nj],
                recv_sem=recv_sems.at[slot, nj],
                device_id=peer,
                device_id_type=pl.DeviceIdType.MESH,
            )

        rdmas = [[None] * NCH for _ in range(B)]
        out_dmas = [None] * B
        for b in range(B):
            slot = b % 2

            wait_all(load_theirs(b))
            if b >= 2:
                for r in rdmas[b - 2]:
                    r.wait_send()
            xt = x_tiles[0].astype(jnp.bfloat16)
            for nj in range(NCH):
                cols = slice(nj * N_CHUNK, (nj + 1) * N_CHUNK)
                send_buf[slot, :, cols] = jnp.dot(
                    xt, w_ref[:, cols], preferred_element_type=jnp.float32
                ).astype(jnp.bfloat16)
                if b >= 2 and nj == 0:
                    pl.semaphore_wait(credit_sem, 1)
                rdma = chunk_rdma(slot, nj)
                rdma.start()
                rdmas[b][nj] = rdma
            if b + 1 < B:
                start_all(load_theirs(b + 1))

            wait_all(load_mine(b))
            if b >= 2:
                out_dmas[b - 2].wait()
            xm = x_tiles[1].astype(jnp.bfloat16)
            for nj in range(NCH):
                cols = slice(nj * N_CHUNK, (nj + 1) * N_CHUNK)
                out_stage[slot, :, cols] = jnp.dot(
                    xm, w_ref[:, cols], preferred_element_type=jnp.float32
                )
            if b + 1 < B:
                start_all(load_mine(b + 1))

            if b > 0:
                pslot = (b - 1) % 2
                for nj in range(NCH):
                    cols = slice(nj * N_CHUNK, (nj + 1) * N_CHUNK)
                    rdmas[b - 1][nj].wait_recv()
                    out_stage[pslot, :, cols] = (
                        out_stage[pslot, :, cols]
                        + recv_buf[pslot, :, cols].astype(jnp.float32)
                    )
                if b - 1 < B - 2:
                    pl.semaphore_signal(
                        credit_sem, inc=1, device_id=peer,
                        device_id_type=pl.DeviceIdType.MESH,
                    )
                dma = pltpu.make_async_copy(
                    out_stage.at[pslot], out_hbm.at[b - 1], out_sems.at[pslot]
                )
                dma.start()
                out_dmas[b - 1] = dma

        lslot = (B - 1) % 2
        for nj in range(NCH):
            cols = slice(nj * N_CHUNK, (nj + 1) * N_CHUNK)
            rdmas[B - 1][nj].wait_recv()
            out_stage[lslot, :, cols] = (
                out_stage[lslot, :, cols]
                + recv_buf[lslot, :, cols].astype(jnp.float32)
            )
        dma = pltpu.make_async_copy(
            out_stage.at[lslot], out_hbm.at[B - 1], out_sems.at[lslot]
        )
        dma.start()
        out_dmas[B - 2].wait()
        dma.wait()
        for r in rdmas[B - 2]:
            r.wait_send()
        for r in rdmas[B - 1]:
            r.wait_send()

    return pl.pallas_call(
        body,
        out_shape=jax.ShapeDtypeStruct((B, S_HALF, N), jnp.float32),
        in_specs=[
            pl.BlockSpec(memory_space=pltpu.MemorySpace.HBM),
            pl.BlockSpec(memory_space=pltpu.VMEM),
        ],
        out_specs=pl.BlockSpec(memory_space=pltpu.MemorySpace.HBM),
        scratch_shapes=[
            pltpu.VMEM((2, S_HALF, K), jnp.float32),
            pltpu.VMEM((2, S_HALF, N), jnp.bfloat16),
            pltpu.VMEM((2, S_HALF, N), jnp.bfloat16),
            pltpu.VMEM((2, S_HALF, N), jnp.float32),
            pltpu.SemaphoreType.DMA((2,)),
            pltpu.SemaphoreType.DMA((2, N // N_CHUNK)),
            pltpu.SemaphoreType.DMA((2, N // N_CHUNK)),
            pltpu.SemaphoreType.DMA((2,)),
            pltpu.SemaphoreType.REGULAR,
        ],
        compiler_params=pltpu.CompilerParams(
            collective_id=0,
            vmem_limit_bytes=96 * 1024 * 1024,
        ),
    )(O, w)


# device time: 236483 ns/iter; 2.7621x vs baseline; 1.0094x over previous
import jax
import jax.numpy as jnp
from jax import lax
from jax.experimental import pallas as pl
from jax.experimental.pallas import tpu as pltpu

B, S, H_LOC, D = 4, 1024, 16, 128
K = H_LOC * D
N = 4096
S_HALF = S // 2
N_CHUNK = 1024


def kernel(O, Wo):
    w = Wo.astype(jnp.bfloat16)

    def body(x_hbm, w_ref, out_ref, x_tiles, send_buf, recv_buf,
             x_sems, send_sems, recv_sems, credit_sem):
        my_x = lax.axis_index("x")
        my_y = lax.axis_index("y")
        my_z = lax.axis_index("z")
        peer = (1 - my_x, my_y, my_z)

        def load_half(b, which, slot):
            s0 = which * S_HALF
            return [
                pltpu.make_async_copy(
                    x_hbm.at[b, pl.ds(s0, S_HALF), h, :],
                    x_tiles.at[slot, :, pl.ds(h * D, D)],
                    x_sems.at[slot],
                )
                for h in range(H_LOC)
            ]

        def load_theirs(b):
            return load_half(b, 1 - my_x, 0)

        def load_mine(b):
            return load_half(b, my_x, 1)

        def start_all(copies):
            for c in copies:
                c.start()

        def wait_all(copies):
            for c in copies:
                c.wait()

        start_all(load_theirs(0))
        start_all(load_mine(0))

        barrier_sem = pltpu.get_barrier_semaphore()
        pl.semaphore_signal(
            barrier_sem, inc=1, device_id=peer,
            device_id_type=pl.DeviceIdType.MESH,
        )
        pl.semaphore_wait(barrier_sem, 1)

        NCH = N // N_CHUNK

        def chunk_rdma(slot, nj):
            cols = pl.ds(nj * N_CHUNK, N_CHUNK)
            return pltpu.make_async_remote_copy(
                src_ref=send_buf.at[slot, :, cols],
                dst_ref=recv_buf.at[slot, :, cols],
                send_sem=send_sems.at[slot, nj],
                recv_sem=recv_sems.at[slot, nj],
                device_id=peer,
                device_id_type=pl.DeviceIdType.MESH,
            )

        rdmas = [[None] * NCH for _ in range(B)]
        out_dmas = [None] * B
        for b in range(B):
            slot = b % 2

            wait_all(load_theirs(b))
            if b >= 2:
                for r in rdmas[b - 2]:
                    r.wait_send()
            xt = x_tiles[0].astype(jnp.bfloat16)
            for nj in range(NCH):
                cols = slice(nj * N_CHUNK, (nj + 1) * N_CHUNK)
                send_buf[slot, :, cols] = jnp.dot(
                    xt, w_ref[:, cols], preferred_element_type=jnp.float32
                ).astype(jnp.bfloat16)
                if b >= 2 and nj == 0:
                    pl.semaphore_wait(credit_sem, 1)
                rdma = chunk_rdma(slot, nj)
                rdma.start()
                rdmas[b][nj] = rdma
            if b + 1 < B:
                start_all(load_theirs(b + 1))

            wait_all(load_mine(b))
            xm = x_tiles[1].astype(jnp.bfloat16)
            for nj in range(NCH):
                cols = slice(nj * N_CHUNK, (nj + 1) * N_CHUNK)
                out_ref[b, :, cols] = jnp.dot(
                    xm, w_ref[:, cols], preferred_element_type=jnp.float32
                ).astype(jnp.bfloat16)
            if b + 1 < B:
                start_all(load_mine(b + 1))

            if b > 0:
                pslot = (b - 1) % 2
                for nj in range(NCH):
                    cols = slice(nj * N_CHUNK, (nj + 1) * N_CHUNK)
                    rdmas[b - 1][nj].wait_recv()
                    out_ref[b - 1, :, cols] = (
                        out_ref[b - 1, :, cols] + recv_buf[pslot, :, cols]
                    )
                if b - 1 < B - 2:
                    pl.semaphore_signal(
                        credit_sem, inc=1, device_id=peer,
                        device_id_type=pl.DeviceIdType.MESH,
                    )

        lslot = (B - 1) % 2
        for nj in range(NCH):
            cols = slice(nj * N_CHUNK, (nj + 1) * N_CHUNK)
            rdmas[B - 1][nj].wait_recv()
            out_ref[B - 1, :, cols] = (
                out_ref[B - 1, :, cols] + recv_buf[lslot, :, cols]
            )
        for r in rdmas[B - 2]:
            r.wait_send()
        for r in rdmas[B - 1]:
            r.wait_send()

    out = pl.pallas_call(
        body,
        out_shape=jax.ShapeDtypeStruct((B, S_HALF, N), jnp.bfloat16),
        in_specs=[
            pl.BlockSpec(memory_space=pltpu.MemorySpace.HBM),
            pl.BlockSpec(memory_space=pltpu.VMEM),
        ],
        out_specs=pl.BlockSpec(memory_space=pltpu.VMEM),
        scratch_shapes=[
            pltpu.VMEM((2, S_HALF, K), jnp.float32),
            pltpu.VMEM((2, S_HALF, N), jnp.bfloat16),
            pltpu.VMEM((2, S_HALF, N), jnp.bfloat16),
            pltpu.SemaphoreType.DMA((2,)),
            pltpu.SemaphoreType.DMA((2, N // N_CHUNK)),
            pltpu.SemaphoreType.DMA((2, N // N_CHUNK)),
            pltpu.SemaphoreType.REGULAR,
        ],
        compiler_params=pltpu.CompilerParams(
            collective_id=0,
            vmem_limit_bytes=96 * 1024 * 1024,
        ),
    )(O, w)
    return out.astype(jnp.float32)
